# baseline (speedup 1.0000x reference)
import jax
import jax.numpy as jnp
from jax import lax
from jax.experimental import pallas as pl
from jax.experimental.pallas import tpu as pltpu
from jax.experimental.pallas import tpu_sc as plsc

N = 878
NR = 880
NSQ = NR * NR
CHUNK = NSQ // 16
IO_CHUNK = CHUNK // 5
E_EDGES = 56192
EPW = E_EDGES // 16
EPW_PAD = 3584
NHEAD = 8
HF = 64
NPAIR = 16384
PAIR_PW = NPAIR // 32
PQ_PAD = 880


def _counts_body(eml_h, edl_h, out_ml, out_dl,
                 dstv, srcv, idxv, valv, iobuf, shared):
    cid = lax.axis_index("c")
    sid = lax.axis_index("s")
    lanes16 = lax.iota(jnp.int32, 16)
    zers16 = jnp.zeros((16,), jnp.float32)

    def fillz(i, carry):
        iobuf[pl.ds(i * 16, 16)] = zers16
        return carry

    lax.fori_loop(0, IO_CHUNK // 16, fillz, 0)
    for k in range(CHUNK // IO_CHUNK):
        pltpu.sync_copy(iobuf, shared.at[pl.ds(sid * CHUNK + k * IO_CHUNK,
                                               IO_CHUNK)])

    base_e = sid * EPW

    @pl.when(cid == 0)
    def _():
        pltpu.sync_copy(eml_h.at[pl.ds(E_EDGES + base_e, EPW)],
                        dstv.at[pl.ds(0, EPW)])
        pltpu.sync_copy(eml_h.at[pl.ds(base_e, EPW)], srcv.at[pl.ds(0, EPW)])

    @pl.when(cid == 1)
    def _():
        pltpu.sync_copy(edl_h.at[pl.ds(E_EDGES + base_e, EPW)],
                        dstv.at[pl.ds(0, EPW)])
        pltpu.sync_copy(edl_h.at[pl.ds(base_e, EPW)], srcv.at[pl.ds(0, EPW)])

    ones = jnp.full((16,), 1.0, jnp.float32)

    def step(i, carry):
        o = i * 16
        d16 = dstv[pl.ds(o, 16)]
        s16 = srcv[pl.ds(o, 16)]
        valid = (o + lanes16) < EPW
        idx = jnp.where(valid, d16 * NR + s16, (NR - 1) * NR + lanes16)
        val = jnp.where(valid, ones, zers16)
        idxv[pl.ds(o, 16)] = idx
        valv[pl.ds(o, 16)] = val
        return carry

    lax.fori_loop(0, EPW_PAD // 16, step, 0)
    plsc.subcore_barrier()
    pltpu.sync_copy(valv, shared.at[idxv], add=True)
    plsc.subcore_barrier()

    for k in range(CHUNK // IO_CHUNK):
        off = sid * CHUNK + k * IO_CHUNK
        pltpu.sync_copy(shared.at[pl.ds(off, IO_CHUNK)], iobuf)

        @pl.when(cid == 0)
        def _():
            pltpu.sync_copy(iobuf, out_ml.at[pl.ds(off, IO_CHUNK)])

        @pl.when(cid == 1)
        def _():
            pltpu.sync_copy(iobuf, out_dl.at[pl.ds(off, IO_CHUNK)])


import functools


@functools.cache
def _counts_call():
    return pl.kernel(
        _counts_body,
        out_type=(jax.ShapeDtypeStruct((NSQ,), jnp.float32),
                  jax.ShapeDtypeStruct((NSQ,), jnp.float32)),
        mesh=plsc.VectorSubcoreMesh(core_axis_name="c", subcore_axis_name="s"),
        scratch_types=[
            pltpu.VMEM((EPW_PAD,), jnp.int32),
            pltpu.VMEM((EPW_PAD,), jnp.int32),
            pltpu.VMEM((EPW_PAD,), jnp.int32),
            pltpu.VMEM((EPW_PAD,), jnp.float32),
            pltpu.VMEM((IO_CHUNK,), jnp.float32),
            pltpu.VMEM_SHARED((NSQ,), jnp.float32),
        ],
    )


def _pred_body(p_h, q_h, dis_h, mir_h, out_h, pbuf, qbuf, dbuf, mbuf, obuf):
    cid = lax.axis_index("c")
    sid = lax.axis_index("s")
    wid = sid * 2 + cid
    base = wid * PAIR_PW
    pltpu.sync_copy(p_h, pbuf)
    pltpu.sync_copy(q_h, qbuf)
    pltpu.sync_copy(dis_h.at[pl.ds(base, PAIR_PW)], dbuf)
    pltpu.sync_copy(mir_h.at[pl.ds(base, PAIR_PW)], mbuf)

    def step(j, carry):
        o = j * 16
        di = dbuf[pl.ds(o, 16)]
        mi = mbuf[pl.ds(o, 16)]
        pv = plsc.load_gather(pbuf, [di])
        qv = plsc.load_gather(qbuf, [mi])
        x = pv + qv
        obuf[pl.ds(o, 16)] = 1.0 / (1.0 + jnp.exp(-x))
        return carry

    lax.fori_loop(0, PAIR_PW // 16, step, 0)
    pltpu.sync_copy(obuf, out_h.at[pl.ds(base, PAIR_PW)])


@functools.cache
def _pred_call():
    return pl.kernel(
        _pred_body,
        out_type=jax.ShapeDtypeStruct((NPAIR,), jnp.float32),
        mesh=plsc.VectorSubcoreMesh(core_axis_name="c", subcore_axis_name="s"),
        compiler_params=pltpu.CompilerParams(needs_layout_passes=False),
        scratch_types=[
            pltpu.VMEM((PQ_PAD,), jnp.float32),
            pltpu.VMEM((PQ_PAD,), jnp.float32),
            pltpu.VMEM((PAIR_PW,), jnp.int32),
            pltpu.VMEM((PAIR_PW,), jnp.int32),
            pltpu.VMEM((PAIR_PW,), jnp.float32),
        ],
    )


def _leaky(x, a):
    return jnp.where(x >= 0, x, a * x)


def _elu(x):
    return jnp.where(x > 0, x, jnp.exp(jnp.minimum(x, 0.0)) - 1.0)


def _pna_block(x, C, deg, degc, MW1, MW2, Mb, U1, U2, U3, Ub, mixW, mixb):
    A = jnp.dot(x, MW1, preferred_element_type=jnp.float32)
    B = jnp.dot(x, MW2, preferred_element_type=jnp.float32) + Mb[None, :]
    A2 = A * A
    S1 = jnp.dot(C, A, preferred_element_type=jnp.float32)
    S2 = jnp.dot(C, A2, preferred_element_type=jnp.float32)
    S3 = jnp.dot(C, A2 * A, preferred_element_type=jnp.float32)
    S4 = jnp.dot(C, A2 * A2, preferred_element_type=jnp.float32)
    inv = 1.0 / degc
    mean = (S1 + deg * B) * inv
    c = B - mean
    c2 = c * c
    sum4 = S4 + 4.0 * S3 * c + 6.0 * S2 * c2 + 4.0 * S1 * (c2 * c) + deg * (c2 * c2)
    m4 = jnp.sqrt(jnp.sqrt(jnp.maximum(sum4 * inv, 0.0) + 1e-5))
    h = (jnp.dot(x, U1, preferred_element_type=jnp.float32)
         + jnp.dot(mean, U2, preferred_element_type=jnp.float32)
         + jnp.dot(m4, U3, preferred_element_type=jnp.float32) + Ub[None, :])
    return _leaky(jnp.dot(h, mixW, preferred_element_type=jnp.float32)
                  + mixb[None, :], 0.01)


def _gat_block(x, C, Wg, AelT, Wer, ones_col):
    z = jnp.dot(x, Wg, preferred_element_type=jnp.float32)
    el_t = lax.dot_general(AelT, z, (((0,), (1,)), ((), ())),
                           preferred_element_type=jnp.float32)
    er = jnp.dot(z, Wer, preferred_element_type=jnp.float32)
    outs = []
    for h in range(NHEAD):
        e = _leaky(el_t[h:h + 1, :] + er[:, h:h + 1], 0.2)
        p = C * jnp.exp(e)
        z_aug = jnp.concatenate([z[:, h * HF:(h + 1) * HF], ones_col], axis=1)
        res = jnp.dot(p, z_aug, preferred_element_type=jnp.float32)
        outs.append(res[:, :HF] * (1.0 / (res[:, HF:HF + 1] + 1e-9)))
    return _elu(jnp.concatenate(outs, axis=1))


def _dense_body(feat_ml, feat_dl, C_ml, C_dl, d_sim, m_sim,
                d_fc0_W, d_fc0_b, m_fc0_W, m_fc0_b,
                M_ml_W, M_ml_b, U_ml_W, U_ml_b, mix_ml_W, mix_ml_b,
                M_dl_W, M_dl_b, U_dl_W, U_dl_b, mix_dl_W, mix_dl_b,
                Wg_ml, Wg_dl, attn_pack,
                h_fc_W, h_fc_b, pred_W, pred_b, v1, v2, v3, v4, out_ref):
    x_ml = feat_ml[...]
    x_dl = feat_dl[...]
    Cml = lax.slice(C_ml[...], (0, 0), (N, N))
    Cdl = lax.slice(C_dl[...], (0, 0), (N, N))
    ones_col = jnp.ones((N, 1), jnp.float32)
    deg_ml = jnp.sum(Cml, axis=1, keepdims=True)
    deg_dl = jnp.sum(Cdl, axis=1, keepdims=True)
    degc_ml = jnp.maximum(deg_ml, 1.0)
    degc_dl = jnp.maximum(deg_dl, 1.0)

    Dml, Ddl = 559, 447
    MWml = M_ml_W[...]
    UWml = U_ml_W[...]
    MWdl = M_dl_W[...]
    UWdl = U_dl_W[...]
    pack = attn_pack[...]

    pna_ml = _pna_block(x_ml, Cml, deg_ml, degc_ml,
                        MWml[:Dml], MWml[Dml:], M_ml_b[...],
                        UWml[:Dml], UWml[Dml:2 * Dml], UWml[2 * Dml:],
                        U_ml_b[...], mix_ml_W[...], mix_ml_b[...])
    gat_ml = _gat_block(x_ml, Cml, Wg_ml[...], pack[:, 0:8], pack[:, 8:16],
                        ones_col)
    pna_dl = _pna_block(x_dl, Cdl, deg_dl, degc_dl,
                        MWdl[:Ddl], MWdl[Ddl:], M_dl_b[...],
                        UWdl[:Ddl], UWdl[Ddl:2 * Ddl], UWdl[2 * Ddl:],
                        U_dl_b[...], mix_dl_W[...], mix_dl_b[...])
    gat_dl = _gat_block(x_dl, Cdl, Wg_dl[...], pack[:, 16:24], pack[:, 24:32],
                        ones_col)

    rowmask = lax.broadcasted_iota(jnp.int32, (N, 1), 0) < 383
    combo_dl = v1[0, 0] * pna_dl + v1[1, 0] * gat_dl
    combo_ml = v2[0, 0] * (pna_ml + gat_ml)
    sim_d = jnp.dot(d_sim[...], d_fc0_W[...],
                    preferred_element_type=jnp.float32) + d_fc0_b[...][None, :]
    sim_m = jnp.dot(m_sim[...], m_fc0_W[...],
                    preferred_element_type=jnp.float32) + m_fc0_b[...][None, :]
    sim_f = jnp.concatenate([sim_d, sim_m], axis=0)
    h_pre = (jnp.where(rowmask, v3[0, 0], v4[0, 0])
             * jnp.where(rowmask, combo_dl, combo_ml)
             + jnp.where(rowmask, v3[1, 0], v4[1, 0]) * sim_f)
    h2 = _elu(jnp.dot(h_pre, h_fc_W[...], preferred_element_type=jnp.float32)
              + h_fc_b[...][None, :])
    Wp = pred_W[...]
    Wcat = jnp.concatenate([Wp[:256], Wp[256:]], axis=1)
    bvec = jnp.pad(pred_b[...], (0, 1))
    out_ref[...] = (jnp.dot(h2, Wcat, preferred_element_type=jnp.float32)
                    + bvec[None, :])


def _dense_call(args):
    n_in = len(args)
    specs = [pl.BlockSpec(memory_space=pltpu.VMEM)] * (n_in - 4) + \
            [pl.BlockSpec(memory_space=pltpu.SMEM)] * 4
    return pl.pallas_call(
        _dense_body,
        out_shape=jax.ShapeDtypeStruct((N, 2), jnp.float32),
        in_specs=specs,
        out_specs=pl.BlockSpec(memory_space=pltpu.VMEM),
    )(*args)


def _attn_mat(a):
    eye = jnp.eye(NHEAD, dtype=a.dtype)
    return (a[:, :, None] * eye[:, None, :]).reshape(NHEAD * HF, NHEAD)


def kernel(feat_ml, feat_dl, edge_index_ml, edge_index_dl, d_sim, m_sim,
           diseases, mirnas, v1, v2, v3, v4, M_ml_W, M_ml_b, U_ml_W, U_ml_b,
           mix_ml_W, mix_ml_b, M_dl_W, M_dl_b, U_dl_W, U_dl_b, mix_dl_W,
           mix_dl_b, Wg_ml, a_ml_l, a_ml_r, Wg_dl, a_dl_l, a_dl_r, d_fc0_W,
           d_fc0_b, m_fc0_W, m_fc0_b, h_fc_W, h_fc_b, pred_W, pred_b):
    eml = edge_index_ml.astype(jnp.int32).reshape(2 * E_EDGES)
    edl = edge_index_dl.astype(jnp.int32).reshape(2 * E_EDGES)
    cml_flat, cdl_flat = _counts_call()(eml, edl)
    C_ml = cml_flat.reshape(NR, NR)
    C_dl = cdl_flat.reshape(NR, NR)

    attn_pack = jnp.concatenate(
        [_attn_mat(a_ml_l), _attn_mat(a_ml_r),
         _attn_mat(a_dl_l), _attn_mat(a_dl_r)], axis=1)

    args = [
        feat_ml, feat_dl, C_ml, C_dl, d_sim, m_sim,
        d_fc0_W, d_fc0_b, m_fc0_W, m_fc0_b,
        M_ml_W, M_ml_b, U_ml_W, U_ml_b, mix_ml_W, mix_ml_b,
        M_dl_W, M_dl_b, U_dl_W, U_dl_b, mix_dl_W, mix_dl_b,
        Wg_ml, Wg_dl, attn_pack,
        h_fc_W, h_fc_b, pred_W, pred_b, v1, v2, v3, v4,
    ]
    pq = _dense_call(args)

    p = jnp.pad(pq[:, 0], (0, PQ_PAD - N))
    q = jnp.pad(pq[:, 1], (0, PQ_PAD - N))
    out = _pred_call()(p, q, diseases.astype(jnp.int32),
                       mirnas.astype(jnp.int32))
    return out.reshape(NPAIR, 1)


# speedup vs baseline: 1.0890x; 1.0890x over previous
import jax
import jax.numpy as jnp
from jax import lax
from jax.experimental import pallas as pl
from jax.experimental.pallas import tpu as pltpu
from jax.experimental.pallas import tpu_sc as plsc

N = 878
NR = 880
NSQ = NR * NR
CHUNK = NSQ // 16
IO_CHUNK = CHUNK // 5
E_EDGES = 56192
EPW = E_EDGES // 16
EPW_PAD = 3584
NHEAD = 8
HF = 64
NPAIR = 16384
PAIR_PW = NPAIR // 32
PQ_PAD = 880


def _counts_body(eml_h, edl_h, out_ml, out_dl,
                 dstv, srcv, idxv, valv, iobuf, shared):
    cid = lax.axis_index("c")
    sid = lax.axis_index("s")
    lanes16 = lax.iota(jnp.int32, 16)
    zers16 = jnp.zeros((16,), jnp.float32)

    def fillz(i, carry):
        iobuf[pl.ds(i * 16, 16)] = zers16
        return carry

    lax.fori_loop(0, IO_CHUNK // 16, fillz, 0)
    for k in range(CHUNK // IO_CHUNK):
        pltpu.sync_copy(iobuf, shared.at[pl.ds(sid * CHUNK + k * IO_CHUNK,
                                               IO_CHUNK)])

    base_e = sid * EPW

    @pl.when(cid == 0)
    def _():
        pltpu.sync_copy(eml_h.at[pl.ds(E_EDGES + base_e, EPW)],
                        dstv.at[pl.ds(0, EPW)])
        pltpu.sync_copy(eml_h.at[pl.ds(base_e, EPW)], srcv.at[pl.ds(0, EPW)])

    @pl.when(cid == 1)
    def _():
        pltpu.sync_copy(edl_h.at[pl.ds(E_EDGES + base_e, EPW)],
                        dstv.at[pl.ds(0, EPW)])
        pltpu.sync_copy(edl_h.at[pl.ds(base_e, EPW)], srcv.at[pl.ds(0, EPW)])

    ones = jnp.full((16,), 1.0, jnp.float32)

    def step(i, carry):
        o = i * 16
        d16 = dstv[pl.ds(o, 16)]
        s16 = srcv[pl.ds(o, 16)]
        valid = (o + lanes16) < EPW
        idx = jnp.where(valid, d16 * NR + s16, (NR - 1) * NR + lanes16)
        val = jnp.where(valid, ones, zers16)
        idxv[pl.ds(o, 16)] = idx
        valv[pl.ds(o, 16)] = val
        return carry

    lax.fori_loop(0, EPW_PAD // 16, step, 0)
    plsc.subcore_barrier()
    pltpu.sync_copy(valv, shared.at[idxv], add=True)
    plsc.subcore_barrier()

    for k in range(CHUNK // IO_CHUNK):
        off = sid * CHUNK + k * IO_CHUNK
        pltpu.sync_copy(shared.at[pl.ds(off, IO_CHUNK)], iobuf)

        @pl.when(cid == 0)
        def _():
            pltpu.sync_copy(iobuf, out_ml.at[pl.ds(off, IO_CHUNK)])

        @pl.when(cid == 1)
        def _():
            pltpu.sync_copy(iobuf, out_dl.at[pl.ds(off, IO_CHUNK)])


import functools


@functools.cache
def _counts_call():
    return pl.kernel(
        _counts_body,
        out_type=(jax.ShapeDtypeStruct((NSQ,), jnp.float32),
                  jax.ShapeDtypeStruct((NSQ,), jnp.float32)),
        mesh=plsc.VectorSubcoreMesh(core_axis_name="c", subcore_axis_name="s"),
        scratch_types=[
            pltpu.VMEM((EPW_PAD,), jnp.int32),
            pltpu.VMEM((EPW_PAD,), jnp.int32),
            pltpu.VMEM((EPW_PAD,), jnp.int32),
            pltpu.VMEM((EPW_PAD,), jnp.float32),
            pltpu.VMEM((IO_CHUNK,), jnp.float32),
            pltpu.VMEM_SHARED((NSQ,), jnp.float32),
        ],
    )


def _pred_body(p_h, q_h, dis_h, mir_h, out_h, pbuf, qbuf, dbuf, mbuf, obuf):
    cid = lax.axis_index("c")
    sid = lax.axis_index("s")
    wid = sid * 2 + cid
    base = wid * PAIR_PW
    pltpu.sync_copy(p_h, pbuf)
    pltpu.sync_copy(q_h, qbuf)
    pltpu.sync_copy(dis_h.at[pl.ds(base, PAIR_PW)], dbuf)
    pltpu.sync_copy(mir_h.at[pl.ds(base, PAIR_PW)], mbuf)

    def step(j, carry):
        o = j * 16
        di = dbuf[pl.ds(o, 16)]
        mi = mbuf[pl.ds(o, 16)]
        pv = plsc.load_gather(pbuf, [di])
        qv = plsc.load_gather(qbuf, [mi])
        x = pv + qv
        obuf[pl.ds(o, 16)] = 1.0 / (1.0 + jnp.exp(-x))
        return carry

    lax.fori_loop(0, PAIR_PW // 16, step, 0)
    pltpu.sync_copy(obuf, out_h.at[pl.ds(base, PAIR_PW)])


@functools.cache
def _pred_call():
    return pl.kernel(
        _pred_body,
        out_type=jax.ShapeDtypeStruct((NPAIR,), jnp.float32),
        mesh=plsc.VectorSubcoreMesh(core_axis_name="c", subcore_axis_name="s"),
        compiler_params=pltpu.CompilerParams(needs_layout_passes=False),
        scratch_types=[
            pltpu.VMEM((PQ_PAD,), jnp.float32),
            pltpu.VMEM((PQ_PAD,), jnp.float32),
            pltpu.VMEM((PAIR_PW,), jnp.int32),
            pltpu.VMEM((PAIR_PW,), jnp.int32),
            pltpu.VMEM((PAIR_PW,), jnp.float32),
        ],
    )


def _leaky(x, a):
    return jnp.where(x >= 0, x, a * x)


def _elu(x):
    return jnp.where(x > 0, x, jnp.exp(jnp.minimum(x, 0.0)) - 1.0)


def _pna_block(x, C, deg, degc, MW1, MW2, Mb, U1, U2, U3, Ub, mixW, mixb):
    A = jnp.dot(x, MW1, preferred_element_type=jnp.float32)
    B = jnp.dot(x, MW2, preferred_element_type=jnp.float32) + Mb[None, :]
    A2 = A * A
    S1 = jnp.dot(C, A, preferred_element_type=jnp.float32)
    S2 = jnp.dot(C, A2, preferred_element_type=jnp.float32)
    S3 = jnp.dot(C, A2 * A, preferred_element_type=jnp.float32)
    S4 = jnp.dot(C, A2 * A2, preferred_element_type=jnp.float32)
    inv = 1.0 / degc
    mean = (S1 + deg * B) * inv
    c = B - mean
    c2 = c * c
    sum4 = S4 + 4.0 * S3 * c + 6.0 * S2 * c2 + 4.0 * S1 * (c2 * c) + deg * (c2 * c2)
    m4 = jnp.sqrt(jnp.sqrt(jnp.maximum(sum4 * inv, 0.0) + 1e-5))
    h = (jnp.dot(x, U1, preferred_element_type=jnp.float32)
         + jnp.dot(mean, U2, preferred_element_type=jnp.float32)
         + jnp.dot(m4, U3, preferred_element_type=jnp.float32) + Ub[None, :])
    return _leaky(jnp.dot(h, mixW, preferred_element_type=jnp.float32)
                  + mixb[None, :], 0.01)


def _gat_block(x, C, Wg, AelT, Wer, ones_col):
    z = jnp.dot(x, Wg, preferred_element_type=jnp.float32)
    el_t = lax.dot_general(AelT, z, (((0,), (1,)), ((), ())),
                           preferred_element_type=jnp.float32)
    er = jnp.dot(z, Wer, preferred_element_type=jnp.float32)
    outs = []
    for h in range(NHEAD):
        e = _leaky(el_t[h:h + 1, :] + er[:, h:h + 1], 0.2)
        p = C * jnp.exp(e)
        z_aug = jnp.concatenate([z[:, h * HF:(h + 1) * HF], ones_col], axis=1)
        res = jnp.dot(p, z_aug, preferred_element_type=jnp.float32)
        outs.append(res[:, :HF] * (1.0 / (res[:, HF:HF + 1] + 1e-9)))
    return _elu(jnp.concatenate(outs, axis=1))


def _dense_body(feat_ml, feat_dl, C_ml, C_dl, d_sim, m_sim,
                d_fc0_W, d_fc0_b, m_fc0_W, m_fc0_b,
                M_ml_W, M_ml_b, U_ml_W, U_ml_b, mix_ml_W, mix_ml_b,
                M_dl_W, M_dl_b, U_dl_W, U_dl_b, mix_dl_W, mix_dl_b,
                Wg_ml, Wg_dl, attn_pack,
                h_fc_W, h_fc_b, pred_W, pred_b, dis2, mir2,
                v1, v2, v3, v4, out_ref):
    x_ml = feat_ml[...]
    x_dl = feat_dl[...]
    Cml = lax.slice(C_ml[...], (0, 0), (N, N))
    Cdl = lax.slice(C_dl[...], (0, 0), (N, N))
    ones_col = jnp.ones((N, 1), jnp.float32)
    deg_ml = jnp.sum(Cml, axis=1, keepdims=True)
    deg_dl = jnp.sum(Cdl, axis=1, keepdims=True)
    degc_ml = jnp.maximum(deg_ml, 1.0)
    degc_dl = jnp.maximum(deg_dl, 1.0)

    Dml, Ddl = 559, 447
    MWml = M_ml_W[...]
    UWml = U_ml_W[...]
    MWdl = M_dl_W[...]
    UWdl = U_dl_W[...]
    pack = attn_pack[...]

    pna_ml = _pna_block(x_ml, Cml, deg_ml, degc_ml,
                        MWml[:Dml], MWml[Dml:], M_ml_b[...],
                        UWml[:Dml], UWml[Dml:2 * Dml], UWml[2 * Dml:],
                        U_ml_b[...], mix_ml_W[...], mix_ml_b[...])
    gat_ml = _gat_block(x_ml, Cml, Wg_ml[...], pack[:, 0:8], pack[:, 8:16],
                        ones_col)
    pna_dl = _pna_block(x_dl, Cdl, deg_dl, degc_dl,
                        MWdl[:Ddl], MWdl[Ddl:], M_dl_b[...],
                        UWdl[:Ddl], UWdl[Ddl:2 * Ddl], UWdl[2 * Ddl:],
                        U_dl_b[...], mix_dl_W[...], mix_dl_b[...])
    gat_dl = _gat_block(x_dl, Cdl, Wg_dl[...], pack[:, 16:24], pack[:, 24:32],
                        ones_col)

    rowmask = lax.broadcasted_iota(jnp.int32, (N, 1), 0) < 383
    combo_dl = v1[0, 0] * pna_dl + v1[1, 0] * gat_dl
    combo_ml = v2[0, 0] * (pna_ml + gat_ml)
    sim_d = jnp.dot(d_sim[...], d_fc0_W[...],
                    preferred_element_type=jnp.float32) + d_fc0_b[...][None, :]
    sim_m = jnp.dot(m_sim[...], m_fc0_W[...],
                    preferred_element_type=jnp.float32) + m_fc0_b[...][None, :]
    sim_f = jnp.concatenate([sim_d, sim_m], axis=0)
    h_pre = (jnp.where(rowmask, v3[0, 0], v4[0, 0])
             * jnp.where(rowmask, combo_dl, combo_ml)
             + jnp.where(rowmask, v3[1, 0], v4[1, 0]) * sim_f)
    h2 = _elu(jnp.dot(h_pre, h_fc_W[...], preferred_element_type=jnp.float32)
              + h_fc_b[...][None, :])
    Wp = pred_W[...]
    Wcat = jnp.concatenate([Wp[:256], Wp[256:]], axis=1)
    bvec = jnp.pad(pred_b[...], (0, 1))
    pq = (jnp.dot(h2, Wcat, preferred_element_type=jnp.float32)
          + bvec[None, :])
    pqpad = jnp.pad(pq, ((0, 7 * 128 - N), (0, 0)))
    dis = dis2[...]
    mir = mir2[...]
    dd, dm = dis // 128, dis & 127
    md, mm = mir // 128, mir & 127
    x = jnp.zeros((128, 128), jnp.float32)
    for t in range(7):
        prow = lax.slice(pqpad, (t * 128, 0), ((t + 1) * 128, 1))
        qrow = lax.slice(pqpad, (t * 128, 1), ((t + 1) * 128, 2))
        pmat = jnp.broadcast_to(prow.reshape(1, 128), (128, 128))
        qmat = jnp.broadcast_to(qrow.reshape(1, 128), (128, 128))
        x = x + jnp.where(dd == t, jnp.take_along_axis(pmat, dm, axis=1), 0.0)
        x = x + jnp.where(md == t, jnp.take_along_axis(qmat, mm, axis=1), 0.0)
    out_ref[...] = 1.0 / (1.0 + jnp.exp(-x))


def _dense_call(args):
    n_in = len(args)
    specs = [pl.BlockSpec(memory_space=pltpu.VMEM)] * (n_in - 4) + \
            [pl.BlockSpec(memory_space=pltpu.SMEM)] * 4
    return pl.pallas_call(
        _dense_body,
        out_shape=jax.ShapeDtypeStruct((128, 128), jnp.float32),
        in_specs=specs,
        out_specs=pl.BlockSpec(memory_space=pltpu.VMEM),
    )(*args)


def _attn_mat(a):
    eye = jnp.eye(NHEAD, dtype=a.dtype)
    return (a[:, :, None] * eye[:, None, :]).reshape(NHEAD * HF, NHEAD)


def kernel(feat_ml, feat_dl, edge_index_ml, edge_index_dl, d_sim, m_sim,
           diseases, mirnas, v1, v2, v3, v4, M_ml_W, M_ml_b, U_ml_W, U_ml_b,
           mix_ml_W, mix_ml_b, M_dl_W, M_dl_b, U_dl_W, U_dl_b, mix_dl_W,
           mix_dl_b, Wg_ml, a_ml_l, a_ml_r, Wg_dl, a_dl_l, a_dl_r, d_fc0_W,
           d_fc0_b, m_fc0_W, m_fc0_b, h_fc_W, h_fc_b, pred_W, pred_b):
    eml = edge_index_ml.astype(jnp.int32).reshape(2 * E_EDGES)
    edl = edge_index_dl.astype(jnp.int32).reshape(2 * E_EDGES)
    cml_flat, cdl_flat = _counts_call()(eml, edl)
    C_ml = cml_flat.reshape(NR, NR)
    C_dl = cdl_flat.reshape(NR, NR)

    attn_pack = jnp.concatenate(
        [_attn_mat(a_ml_l), _attn_mat(a_ml_r),
         _attn_mat(a_dl_l), _attn_mat(a_dl_r)], axis=1)

    args = [
        feat_ml, feat_dl, C_ml, C_dl, d_sim, m_sim,
        d_fc0_W, d_fc0_b, m_fc0_W, m_fc0_b,
        M_ml_W, M_ml_b, U_ml_W, U_ml_b, mix_ml_W, mix_ml_b,
        M_dl_W, M_dl_b, U_dl_W, U_dl_b, mix_dl_W, mix_dl_b,
        Wg_ml, Wg_dl, attn_pack,
        h_fc_W, h_fc_b, pred_W, pred_b,
        diseases.astype(jnp.int32).reshape(128, 128),
        mirnas.astype(jnp.int32).reshape(128, 128),
        v1, v2, v3, v4,
    ]
    out = _dense_call(args)
    return out.reshape(NPAIR, 1)


# speedup vs baseline: 1.0897x; 1.0006x over previous
import jax
import jax.numpy as jnp
from jax import lax
from jax.experimental import pallas as pl
from jax.experimental.pallas import tpu as pltpu
from jax.experimental.pallas import tpu_sc as plsc

N = 878
NR = 880
NSQ = NR * NR
CHUNK = NSQ // 16
IO_CHUNK = CHUNK // 5
E_EDGES = 56192
EPW = E_EDGES // 16
EPW_PAD = 3584
NHEAD = 8
HF = 64
NPAIR = 16384


def _counts_body(eml_h, edl_h, out_ml, out_dl,
                 dstv, srcv, idxv, valv, iobuf, shared):
    cid = lax.axis_index("c")
    sid = lax.axis_index("s")
    lanes16 = lax.iota(jnp.int32, 16)
    zers16 = jnp.zeros((16,), jnp.float32)

    def fillz(i, carry):
        iobuf[pl.ds(i * 16, 16)] = zers16
        return carry

    lax.fori_loop(0, IO_CHUNK // 16, fillz, 0)
    for k in range(CHUNK // IO_CHUNK):
        pltpu.sync_copy(iobuf, shared.at[pl.ds(sid * CHUNK + k * IO_CHUNK,
                                               IO_CHUNK)])

    base_e = sid * EPW

    @pl.when(cid == 0)
    def _():
        pltpu.sync_copy(eml_h.at[pl.ds(E_EDGES + base_e, EPW)],
                        dstv.at[pl.ds(0, EPW)])
        pltpu.sync_copy(eml_h.at[pl.ds(base_e, EPW)], srcv.at[pl.ds(0, EPW)])

    @pl.when(cid == 1)
    def _():
        pltpu.sync_copy(edl_h.at[pl.ds(E_EDGES + base_e, EPW)],
                        dstv.at[pl.ds(0, EPW)])
        pltpu.sync_copy(edl_h.at[pl.ds(base_e, EPW)], srcv.at[pl.ds(0, EPW)])

    ones = jnp.full((16,), 1.0, jnp.float32)

    def step(i, carry):
        o = i * 16
        d16 = dstv[pl.ds(o, 16)]
        s16 = srcv[pl.ds(o, 16)]
        valid = (o + lanes16) < EPW
        idx = jnp.where(valid, d16 * NR + s16, (NR - 1) * NR + lanes16)
        val = jnp.where(valid, ones, zers16)
        idxv[pl.ds(o, 16)] = idx
        valv[pl.ds(o, 16)] = val
        return carry

    lax.fori_loop(0, EPW_PAD // 16, step, 0)
    plsc.subcore_barrier()
    pltpu.sync_copy(valv, shared.at[idxv], add=True)
    plsc.subcore_barrier()

    for k in range(CHUNK // IO_CHUNK):
        off = sid * CHUNK + k * IO_CHUNK
        pltpu.sync_copy(shared.at[pl.ds(off, IO_CHUNK)], iobuf)

        @pl.when(cid == 0)
        def _():
            pltpu.sync_copy(iobuf, out_ml.at[pl.ds(off, IO_CHUNK)])

        @pl.when(cid == 1)
        def _():
            pltpu.sync_copy(iobuf, out_dl.at[pl.ds(off, IO_CHUNK)])


import functools


@functools.cache
def _counts_call():
    return pl.kernel(
        _counts_body,
        out_type=(jax.ShapeDtypeStruct((NSQ,), jnp.float32),
                  jax.ShapeDtypeStruct((NSQ,), jnp.float32)),
        mesh=plsc.VectorSubcoreMesh(core_axis_name="c", subcore_axis_name="s"),
        scratch_types=[
            pltpu.VMEM((EPW_PAD,), jnp.int32),
            pltpu.VMEM((EPW_PAD,), jnp.int32),
            pltpu.VMEM((EPW_PAD,), jnp.int32),
            pltpu.VMEM((EPW_PAD,), jnp.float32),
            pltpu.VMEM((IO_CHUNK,), jnp.float32),
            pltpu.VMEM_SHARED((NSQ,), jnp.float32),
        ],
    )


def _leaky(x, a):
    return jnp.where(x >= 0, x, a * x)


def _elu(x):
    return jnp.where(x > 0, x, jnp.exp(jnp.minimum(x, 0.0)) - 1.0)


def _pna_block(x, C, deg, degc, MW1, MW2, Mb, U1, U2, U3, Ub, mixW, mixb):
    A = jnp.dot(x, MW1, preferred_element_type=jnp.float32)
    B = jnp.dot(x, MW2, preferred_element_type=jnp.float32) + Mb[None, :]
    A2 = A * A
    S1 = jnp.dot(C, A, preferred_element_type=jnp.float32)
    S2 = jnp.dot(C, A2, preferred_element_type=jnp.float32)
    S3 = jnp.dot(C, A2 * A, preferred_element_type=jnp.float32)
    S4 = jnp.dot(C, A2 * A2, preferred_element_type=jnp.float32)
    inv = 1.0 / degc
    mean = (S1 + deg * B) * inv
    c = B - mean
    c2 = c * c
    sum4 = S4 + 4.0 * S3 * c + 6.0 * S2 * c2 + 4.0 * S1 * (c2 * c) + deg * (c2 * c2)
    m4 = jnp.sqrt(jnp.sqrt(jnp.maximum(sum4 * inv, 0.0) + 1e-5))
    h = (jnp.dot(x, U1, preferred_element_type=jnp.float32)
         + jnp.dot(mean, U2, preferred_element_type=jnp.float32)
         + jnp.dot(m4, U3, preferred_element_type=jnp.float32) + Ub[None, :])
    return _leaky(jnp.dot(h, mixW, preferred_element_type=jnp.float32)
                  + mixb[None, :], 0.01)


def _gat_block(x, C, Wg, AelT, Wer, ones_col):
    z = jnp.dot(x, Wg, preferred_element_type=jnp.float32)
    el_t = lax.dot_general(AelT, z, (((0,), (1,)), ((), ())),
                           preferred_element_type=jnp.float32)
    er = jnp.dot(z, Wer, preferred_element_type=jnp.float32)
    outs = []
    for h in range(NHEAD):
        e = _leaky(el_t[h:h + 1, :] + er[:, h:h + 1], 0.2)
        p = C * jnp.exp(e)
        z_aug = jnp.concatenate([z[:, h * HF:(h + 1) * HF], ones_col], axis=1)
        res = jnp.dot(p, z_aug, preferred_element_type=jnp.float32)
        outs.append(res[:, :HF] * (1.0 / (res[:, HF:HF + 1] + 1e-9)))
    return _elu(jnp.concatenate(outs, axis=1))


def _dense_body(feat_ml, feat_dl, C_ml, C_dl, d_sim, m_sim,
                d_fc0_W, d_fc0_b, m_fc0_W, m_fc0_b,
                M_ml_W, M_ml_b, U_ml_W, U_ml_b, mix_ml_W, mix_ml_b,
                M_dl_W, M_dl_b, U_dl_W, U_dl_b, mix_dl_W, mix_dl_b,
                Wg_ml, Wg_dl, attn_pack,
                h_fc_W, h_fc_b, pred_W, pred_b, dis2, mir2,
                v1, v2, v3, v4, out_ref):
    x_ml = feat_ml[...]
    x_dl = feat_dl[...]
    Cml = lax.slice(C_ml[...], (0, 0), (N, N))
    Cdl = lax.slice(C_dl[...], (0, 0), (N, N))
    ones_col = jnp.ones((N, 1), jnp.float32)
    deg_ml = jnp.sum(Cml, axis=1, keepdims=True)
    deg_dl = jnp.sum(Cdl, axis=1, keepdims=True)
    degc_ml = jnp.maximum(deg_ml, 1.0)
    degc_dl = jnp.maximum(deg_dl, 1.0)

    Dml, Ddl = 559, 447
    MWml = M_ml_W[...]
    UWml = U_ml_W[...]
    MWdl = M_dl_W[...]
    UWdl = U_dl_W[...]
    pack = attn_pack[...]

    pna_ml = _pna_block(x_ml, Cml, deg_ml, degc_ml,
                        MWml[:Dml], MWml[Dml:], M_ml_b[...],
                        UWml[:Dml], UWml[Dml:2 * Dml], UWml[2 * Dml:],
                        U_ml_b[...], mix_ml_W[...], mix_ml_b[...])
    gat_ml = _gat_block(x_ml, Cml, Wg_ml[...], pack[:, 0:8], pack[:, 8:16],
                        ones_col)
    pna_dl = _pna_block(x_dl, Cdl, deg_dl, degc_dl,
                        MWdl[:Ddl], MWdl[Ddl:], M_dl_b[...],
                        UWdl[:Ddl], UWdl[Ddl:2 * Ddl], UWdl[2 * Ddl:],
                        U_dl_b[...], mix_dl_W[...], mix_dl_b[...])
    gat_dl = _gat_block(x_dl, Cdl, Wg_dl[...], pack[:, 16:24], pack[:, 24:32],
                        ones_col)

    rowmask = lax.broadcasted_iota(jnp.int32, (N, 1), 0) < 383
    combo_dl = v1[0, 0] * pna_dl + v1[1, 0] * gat_dl
    combo_ml = v2[0, 0] * (pna_ml + gat_ml)
    sim_d = jnp.dot(d_sim[...], d_fc0_W[...],
                    preferred_element_type=jnp.float32) + d_fc0_b[...][None, :]
    sim_m = jnp.dot(m_sim[...], m_fc0_W[...],
                    preferred_element_type=jnp.float32) + m_fc0_b[...][None, :]
    sim_f = jnp.concatenate([sim_d, sim_m], axis=0)
    h_pre = (jnp.where(rowmask, v3[0, 0], v4[0, 0])
             * jnp.where(rowmask, combo_dl, combo_ml)
             + jnp.where(rowmask, v3[1, 0], v4[1, 0]) * sim_f)
    h2 = _elu(jnp.dot(h_pre, h_fc_W[...], preferred_element_type=jnp.float32)
              + h_fc_b[...][None, :])
    Wp = pred_W[...]
    Wcat = jnp.concatenate([Wp[:256], Wp[256:]], axis=1)
    bvec = jnp.pad(pred_b[...], (0, 1))
    pq = (jnp.dot(h2, Wcat, preferred_element_type=jnp.float32)
          + bvec[None, :])
    pqpad = jnp.pad(pq, ((0, 7 * 128 - N), (0, 0)))
    dis = dis2[...]
    mir = mir2[...]
    dd, dm = dis // 128, dis & 127
    md, mm = mir // 128, mir & 127
    x = jnp.zeros((128, 128), jnp.float32)
    for t in range(7):
        prow = lax.slice(pqpad, (t * 128, 0), ((t + 1) * 128, 1))
        qrow = lax.slice(pqpad, (t * 128, 1), ((t + 1) * 128, 2))
        pmat = jnp.broadcast_to(prow.reshape(1, 128), (128, 128))
        qmat = jnp.broadcast_to(qrow.reshape(1, 128), (128, 128))
        x = x + jnp.where(dd == t, jnp.take_along_axis(pmat, dm, axis=1), 0.0)
        x = x + jnp.where(md == t, jnp.take_along_axis(qmat, mm, axis=1), 0.0)
    out_ref[...] = 1.0 / (1.0 + jnp.exp(-x))


def _dense_call(args):
    n_in = len(args)
    specs = [pl.BlockSpec(memory_space=pltpu.VMEM)] * (n_in - 4) + \
            [pl.BlockSpec(memory_space=pltpu.SMEM)] * 4
    return pl.pallas_call(
        _dense_body,
        out_shape=jax.ShapeDtypeStruct((128, 128), jnp.float32),
        in_specs=specs,
        out_specs=pl.BlockSpec(memory_space=pltpu.VMEM),
    )(*args)


def _attn_mat(a):
    eye = jnp.eye(NHEAD, dtype=a.dtype)
    return (a[:, :, None] * eye[:, None, :]).reshape(NHEAD * HF, NHEAD)


def kernel(feat_ml, feat_dl, edge_index_ml, edge_index_dl, d_sim, m_sim,
           diseases, mirnas, v1, v2, v3, v4, M_ml_W, M_ml_b, U_ml_W, U_ml_b,
           mix_ml_W, mix_ml_b, M_dl_W, M_dl_b, U_dl_W, U_dl_b, mix_dl_W,
           mix_dl_b, Wg_ml, a_ml_l, a_ml_r, Wg_dl, a_dl_l, a_dl_r, d_fc0_W,
           d_fc0_b, m_fc0_W, m_fc0_b, h_fc_W, h_fc_b, pred_W, pred_b):
    eml = edge_index_ml.astype(jnp.int32).reshape(2 * E_EDGES)
    edl = edge_index_dl.astype(jnp.int32).reshape(2 * E_EDGES)
    cml_flat, cdl_flat = _counts_call()(eml, edl)
    C_ml = cml_flat.reshape(NR, NR)
    C_dl = cdl_flat.reshape(NR, NR)

    attn_pack = jnp.concatenate(
        [_attn_mat(a_ml_l), _attn_mat(a_ml_r),
         _attn_mat(a_dl_l), _attn_mat(a_dl_r)], axis=1)

    args = [
        feat_ml, feat_dl, C_ml, C_dl, d_sim, m_sim,
        d_fc0_W, d_fc0_b, m_fc0_W, m_fc0_b,
        M_ml_W, M_ml_b, U_ml_W, U_ml_b, mix_ml_W, mix_ml_b,
        M_dl_W, M_dl_b, U_dl_W, U_dl_b, mix_dl_W, mix_dl_b,
        Wg_ml, Wg_dl, attn_pack,
        h_fc_W, h_fc_b, pred_W, pred_b,
        diseases.astype(jnp.int32).reshape(128, 128),
        mirnas.astype(jnp.int32).reshape(128, 128),
        v1, v2, v3, v4,
    ]
    out = _dense_call(args)
    return out.reshape(NPAIR, 1)
